# baseline (speedup 1.0000x reference)
import jax
import jax.numpy as jnp
from jax import lax
from jax.experimental import pallas as pl
from jax.experimental.pallas import tpu as pltpu
from jax.experimental.pallas import tpu_sc as plsc

_B = 16384
_D = 32
_L = 16

_INFO = plsc.get_sparse_core_info()
_NC = _INFO.num_cores
_NS = _INFO.num_subcores
_NW = _NC * _NS
_BPW = _B // _NW
_GROUPS = _BPW // _L
_CHUNK = 128
_NCHUNK = _BPW // _CHUNK


def _sc_body(idx1_hbm, idx2_hbm, emb_hbm, reff_hbm, beta_hbm, out_hbm,
             idx1_v, idx2_v, z1_v, z2_v, r1_v, r2_v, beta_v, out_v, sem):
    wid = lax.axis_index("s") * _NC + lax.axis_index("c")
    base = wid * _BPW

    pltpu.sync_copy(idx1_hbm.at[pl.ds(base, _BPW)], idx1_v)
    pltpu.sync_copy(idx2_hbm.at[pl.ds(base, _BPW)], idx2_v)
    pltpu.sync_copy(beta_hbm, beta_v)

    copies = []
    for k in range(_NCHUNK):
        s = pl.ds(k * _CHUNK, _CHUNK)
        copies.append(pltpu.async_copy(emb_hbm.at[idx1_v.at[s]], z1_v.at[s, :], sem))
        copies.append(pltpu.async_copy(emb_hbm.at[idx2_v.at[s]], z2_v.at[s, :], sem))
        copies.append(pltpu.async_copy(reff_hbm.at[idx1_v.at[s]], r1_v.at[s], sem))
        copies.append(pltpu.async_copy(reff_hbm.at[idx2_v.at[s]], r2_v.at[s], sem))
    for c in copies:
        c.wait()

    beta = beta_v[...]
    lane = lax.iota(jnp.int32, _L)

    def group(g, carry):
        rows = g * _L + lane
        acc = jnp.zeros((_L,), jnp.float32)
        for d in range(_D):
            col = jnp.full((_L,), d, jnp.int32)
            a = plsc.load_gather(z1_v, [rows, col])
            b = plsc.load_gather(z2_v, [rows, col])
            diff = a - b
            acc = acc + diff * diff
        i = plsc.bitcast(acc, jnp.int32)
        i = jnp.int32(0x5F3759DF) - (i >> 1)
        y = plsc.bitcast(i, jnp.float32)
        for _ in range(3):
            y = y * (1.5 - 0.5 * acc * y * y)
        dist = jnp.where(acc > 1e-35, acc * y, 0.0)
        sl = pl.ds(g * _L, _L)
        out_v[sl] = r1_v[sl] + r2_v[sl] - beta * dist
        return carry

    lax.fori_loop(0, _GROUPS, group, 0)
    pltpu.sync_copy(out_v, out_hbm.at[pl.ds(base, _BPW)])


@jax.jit
def _run(p1, p2, emb, reff_flat, beta16):
    ker = pl.kernel(
        _sc_body,
        out_type=jax.ShapeDtypeStruct((_B,), jnp.float32),
        mesh=plsc.VectorSubcoreMesh(core_axis_name="c", subcore_axis_name="s"),
        compiler_params=pltpu.CompilerParams(
            needs_layout_passes=False, use_tc_tiling_on_sc=False),
        scratch_types=[
            pltpu.VMEM((_BPW,), jnp.int32),
            pltpu.VMEM((_BPW,), jnp.int32),
            pltpu.VMEM((_BPW, _D), jnp.float32),
            pltpu.VMEM((_BPW, _D), jnp.float32),
            pltpu.VMEM((_BPW,), jnp.float32),
            pltpu.VMEM((_BPW,), jnp.float32),
            pltpu.VMEM((_L,), jnp.float32),
            pltpu.VMEM((_BPW,), jnp.float32),
            pltpu.SemaphoreType.DMA,
        ],
    )
    return ker(p1, p2, emb, reff_flat, beta16)


def kernel(protein1_idx, protein2_idx, embeddings, random_effects, beta):
    p1 = protein1_idx.astype(jnp.int32)
    p2 = protein2_idx.astype(jnp.int32)
    reff_flat = random_effects.reshape(-1)
    beta16 = jnp.full((_L,), beta, jnp.float32)
    return _run(p1, p2, embeddings, reff_flat, beta16)
